# baseline (speedup 1.0000x reference)
import functools

import jax
import jax.numpy as jnp
from jax import lax
from jax.experimental import pallas as pl
from jax.experimental.pallas import tpu as pltpu
from jax.experimental.pallas import tpu_sc as plsc

N = 1000000
K = 64
B = 16384

NC = 2
NS = 16
L = 16
NW = NC * NS
BPW = B // NW
CHUNK = 128
NCH = BPW // CHUNK
BLOCKS = CHUNK // L

_mesh = plsc.VectorSubcoreMesh(core_axis_name="c", subcore_axis_name="s")


@functools.partial(
    pl.kernel,
    out_type=(
        jax.ShapeDtypeStruct((B,), jnp.float32),
        jax.ShapeDtypeStruct((B,), jnp.float32),
    ),
    mesh=_mesh,
    compiler_params=pltpu.CompilerParams(
        needs_layout_passes=False, use_tc_tiling_on_sc=False),
    scratch_types=[
        pltpu.VMEM((NCH, CHUNK), jnp.int32),
        pltpu.VMEM((NCH, CHUNK), jnp.int32),
        pltpu.VMEM((CHUNK, K), jnp.float32),
        pltpu.VMEM((CHUNK, K), jnp.float32),
        pltpu.VMEM((CHUNK, K), jnp.float32),
        pltpu.VMEM((BPW,), jnp.float32),
        pltpu.VMEM((BPW,), jnp.float32),
        pltpu.SemaphoreType.DMA,
    ],
)
def _mf_sc_kernel(uid_hbm, vid_hbm, U_hbm, ag_hbm, pe_hbm,
                  oag_hbm, ope_hbm,
                  uid_v, vid_v, u_v, ag_v, pe_v, oag_v, ope_v, sem):
    wid = lax.axis_index("s") * NC + lax.axis_index("c")
    base_row = wid * NCH
    pltpu.sync_copy(uid_hbm.at[pl.ds(base_row, NCH)], uid_v)
    pltpu.sync_copy(vid_hbm.at[pl.ds(base_row, NCH)], vid_v)

    for c in range(NCH):
        cu = pltpu.async_copy(U_hbm.at[uid_v.at[c]], u_v, sem)
        ca = pltpu.async_copy(ag_hbm.at[vid_v.at[c]], ag_v, sem)
        cp = pltpu.async_copy(pe_hbm.at[vid_v.at[c]], pe_v, sem)
        cu.wait()
        ca.wait()
        cp.wait()

        def blk_body(b, carry, c=c):
            rows = lax.iota(jnp.int32, L) + b * L
            acc_a = jnp.zeros((L,), jnp.float32)
            acc_p = jnp.zeros((L,), jnp.float32)
            for k in range(K):
                cols = jnp.full((L,), k, jnp.int32)
                u = plsc.load_gather(u_v, [rows, cols])
                a = plsc.load_gather(ag_v, [rows, cols])
                p = plsc.load_gather(pe_v, [rows, cols])
                acc_a = acc_a + u * a
                acc_p = acc_p + u * p
            off = c * CHUNK + b * L
            oag_v[pl.ds(off, L)] = acc_a
            ope_v[pl.ds(off, L)] = acc_p
            return carry

        lax.fori_loop(0, BLOCKS, blk_body, 0)

    base = wid * BPW
    pltpu.sync_copy(oag_v, oag_hbm.at[pl.ds(base, BPW)])
    pltpu.sync_copy(ope_v, ope_hbm.at[pl.ds(base, BPW)])


def kernel(user_id, value_id, U, ag_V, pe_V):
    uid2 = user_id.reshape(B // CHUNK, CHUNK)
    vid2 = value_id.reshape(B // CHUNK, CHUNK)
    return _mf_sc_kernel(uid2, vid2, U, ag_V, pe_V)


# speedup vs baseline: 1.1380x; 1.1380x over previous
import functools

import jax
import jax.numpy as jnp
from jax import lax
from jax.experimental import pallas as pl
from jax.experimental.pallas import tpu as pltpu
from jax.experimental.pallas import tpu_sc as plsc

N = 1000000
K = 64
B = 16384

NC = 2
NS = 16
L = 16
NW = NC * NS
NBKT = (N + 127) // 128
CAP = 2 * B
SB = B + L

_mesh = plsc.VectorSubcoreMesh(core_axis_name="c", subcore_axis_name="s")
_params = pltpu.CompilerParams(needs_layout_passes=False)

_i32 = jnp.int32
_f32 = jnp.float32


@functools.partial(
    pl.kernel,
    out_type=(
        jax.ShapeDtypeStruct((SB, 128), _f32),
        jax.ShapeDtypeStruct((SB, 128), _f32),
        jax.ShapeDtypeStruct((SB, 128), _f32),
    ),
    mesh=_mesh,
    compiler_params=_params,
    scratch_types=[
        pltpu.VMEM((16, 128), _i32),
        pltpu.VMEM((CAP + L,), _i32),
        pltpu.VMEM((CAP + L,), _i32),
        pltpu.VMEM((48,), _i32),
        pltpu.VMEM((K, 128), _f32),
        pltpu.VMEM((K, 128), _f32),
        pltpu.VMEM((K, 128), _f32),
        pltpu.VMEM((K, 128), _f32),
        pltpu.VMEM((K, 128), _f32),
        pltpu.VMEM((K, 128), _f32),
        pltpu.VMEM((L, 128), _f32),
        pltpu.VMEM((L, 128), _f32),
        pltpu.VMEM((L, 128), _f32),
        pltpu.VMEM((1, L), _i32),
        pltpu.VMEM((1, L), _i32),
        pltpu.SemaphoreType.DMA,
        pltpu.SemaphoreType.DMA,
        pltpu.SemaphoreType.DMA,
    ],
)
def _gather_kernel(uid_hbm, vid_hbm, Ut_hbm, agt_hbm, pet_hbm,
                   su_hbm, sa_hbm, sp_hbm,
                   scan_v, keys_v, vals_v, mv_buf,
                   sau, saa, sap, sbu, sba, sbp,
                   fr_u, fr_a, fr_p, fp_u, fp_v,
                   semA, semB, semF):
    wid = lax.axis_index("s") * NC + lax.axis_index("c")
    lo = wid * NBKT // NW
    hi = (wid + 1) * NBKT // NW
    iota = lax.iota(_i32, L)

    cnt = jnp.asarray(0, _i32)
    for ch in range(16):
        tag = 0 if ch < 8 else 1
        src = uid_hbm if ch < 8 else vid_hbm
        row0 = (ch % 8) * 16
        pltpu.sync_copy(src.at[pl.ds(row0, 16)], scan_v)

        def scan_body(g, cnt, tag=tag, row0=row0):
            ids = scan_v[lax.shift_right_logical(g, 3),
                         pl.ds(jnp.bitwise_and(g, 7) * L, L)]
            bkt = lax.shift_right_logical(ids, 7)
            m = (bkt >= lo) & (bkt < hi)
            pos = row0 * 128 + g * L + iota
            val = pos | (tag << 14) | (jnp.bitwise_and(ids, 127) << 15)
            plsc.store_compressed(keys_v.at[pl.ds(cnt, L)], bkt, mask=m)
            plsc.store_compressed(vals_v.at[pl.ds(cnt, L)], val, mask=m)
            return cnt + plsc.all_reduce_population_count(m)[0]

        cnt = lax.fori_loop(0, 128, scan_body, cnt)

    ngrp = lax.div(cnt + (L - 1), jnp.asarray(L, _i32))

    def issue(b, slabs, sem):
        bb = pl.multiple_of(b * 128, 128)
        pltpu.async_copy(Ut_hbm.at[:, pl.ds(bb, 128)], slabs[0], sem)
        pltpu.async_copy(agt_hbm.at[:, pl.ds(bb, 128)], slabs[1], sem)
        pltpu.async_copy(pet_hbm.at[:, pl.ds(bb, 128)], slabs[2], sem)

    def wait_slabs(slabs, sem):
        for s in slabs:
            pltpu.make_async_copy(Ut_hbm.at[:, pl.ds(0, 128)], s, sem).wait()

    def drain_flush():
        pltpu.make_async_copy(fr_u, su_hbm.at[pl.ds(0, L)], semF).wait()
        pltpu.make_async_copy(fr_a, sa_hbm.at[pl.ds(0, L)], semF).wait()
        pltpu.make_async_copy(fr_p, sp_hbm.at[pl.ds(0, L)], semF).wait()

    def extract(slabs, valid_n, out_flag):
        v16 = mv_buf[pl.ds(0, L)]
        valid = iota < valid_n
        pos = jnp.bitwise_and(v16, 16383)
        tagv = jnp.bitwise_and(lax.shift_right_logical(v16, 14), 1)
        col = jnp.bitwise_and(lax.shift_right_logical(v16, 15), 127)
        mu = valid & (tagv == 0)
        mv = valid & (tagv == 1)
        ru = plsc.cumsum(jnp.where(mu, 1, 0)) - 1
        rv = plsc.cumsum(jnp.where(mv, 1, 0)) - 1
        zeros = jnp.zeros((L,), _i32)

        @pl.when(out_flag == 1)
        def _():
            drain_flush()

        fp_u[0, pl.ds(0, L)] = B + iota
        fp_v[0, pl.ds(0, L)] = B + iota
        plsc.store_scatter(fp_u, [zeros, ru], pos, mask=mu)
        plsc.store_scatter(fp_v, [zeros, rv], pos, mask=mv)
        for k in range(K):
            kk = jnp.full((L,), k, _i32)
            gu = plsc.load_gather(slabs[0], [kk, col])
            plsc.store_scatter(fr_u, [ru, kk], gu, mask=mu)
            ga = plsc.load_gather(slabs[1], [kk, col])
            plsc.store_scatter(fr_a, [rv, kk], ga, mask=mv)
            gp = plsc.load_gather(slabs[2], [kk, col])
            plsc.store_scatter(fr_p, [rv, kk], gp, mask=mv)
        pltpu.async_copy(fr_u, su_hbm.at[fp_u.at[0]], semF)
        pltpu.async_copy(fr_a, sa_hbm.at[fp_v.at[0]], semF)
        pltpu.async_copy(fr_p, sp_hbm.at[fp_v.at[0]], semF)

    def process_bucket(b, slabs, out_flag):
        def match_body(g, carry):
            mfill, oflag = carry
            k16 = keys_v[pl.ds(g * L, L)]
            valid = (g * L + iota) < cnt
            m = valid & (k16 == b)
            pc = plsc.all_reduce_population_count(m)[0]

            @pl.when(pc > 0)
            def _():
                plsc.store_compressed(mv_buf.at[pl.ds(mfill, L)],
                                      vals_v[pl.ds(g * L, L)], mask=m)

            nf = mfill + pc

            @pl.when(nf >= L)
            def _():
                extract(slabs, jnp.asarray(L, _i32), oflag)
                rem = mv_buf[pl.ds(L, L)]
                mv_buf[pl.ds(0, L)] = rem

            oflag = jnp.where(nf >= L, 1, oflag)
            mfill = jnp.where(nf >= L, nf - L, nf)
            return (mfill, oflag)

        mfill, oflag = lax.fori_loop(
            0, ngrp, match_body, (jnp.asarray(0, _i32), out_flag))

        @pl.when(mfill > 0)
        def _():
            extract(slabs, mfill, oflag)

        return jnp.where(mfill > 0, jnp.asarray(1, _i32), oflag)

    slabsA = (sau, saa, sap)
    slabsB = (sbu, sba, sbp)

    @pl.when(lo < hi)
    def _():
        issue(lo, slabsA, semA)

    @pl.when(lo + 1 < hi)
    def _():
        issue(lo + 1, slabsB, semB)

    def pair_body(i, out_flag):
        b0 = lo + 2 * i
        b1 = b0 + 1

        wait_slabs(slabsA, semA)
        out_flag = process_bucket(b0, slabsA, out_flag)

        @pl.when(b0 + 2 < hi)
        def _():
            issue(b0 + 2, slabsA, semA)

        @pl.when(b1 < hi)
        def _():
            wait_slabs(slabsB, semB)

        out_flag = process_bucket(b1, slabsB, out_flag)

        @pl.when(b1 + 2 < hi)
        def _():
            issue(b1 + 2, slabsB, semB)

        return out_flag

    npairs = lax.div(hi - lo + 1, jnp.asarray(2, _i32))
    out_flag = lax.fori_loop(0, npairs, pair_body, jnp.asarray(0, _i32))

    @pl.when(out_flag == 1)
    def _():
        drain_flush()


@functools.partial(
    pl.kernel,
    out_type=(
        jax.ShapeDtypeStruct((NW, 4, 128), _f32),
        jax.ShapeDtypeStruct((NW, 4, 128), _f32),
    ),
    mesh=_mesh,
    compiler_params=_params,
    scratch_types=[
        pltpu.VMEM((128, 128), _f32),
        pltpu.VMEM((128, 128), _f32),
        pltpu.VMEM((128, 128), _f32),
        pltpu.VMEM((4, 128), _f32),
        pltpu.VMEM((4, 128), _f32),
        pltpu.SemaphoreType.DMA,
    ],
)
def _dot_kernel(su_hbm, sa_hbm, sp_hbm, oag_hbm, ope_hbm,
                bu, ba, bp, oag_v, ope_v, sem):
    wid = lax.axis_index("s") * NC + lax.axis_index("c")
    base = wid * 512
    iota = lax.iota(_i32, L)
    for c in range(4):
        pltpu.async_copy(su_hbm.at[pl.ds(base + c * 128, 128)], bu, sem)
        pltpu.async_copy(sa_hbm.at[pl.ds(base + c * 128, 128)], ba, sem)
        pltpu.async_copy(sp_hbm.at[pl.ds(base + c * 128, 128)], bp, sem)
        pltpu.make_async_copy(su_hbm.at[pl.ds(0, 128)], bu, sem).wait()
        pltpu.make_async_copy(sa_hbm.at[pl.ds(0, 128)], ba, sem).wait()
        pltpu.make_async_copy(sp_hbm.at[pl.ds(0, 128)], bp, sem).wait()

        def grp_body(g, carry, c=c):
            rows = iota + g * L
            acc_a = jnp.zeros((L,), _f32)
            acc_p = jnp.zeros((L,), _f32)
            for k in range(K):
                kk = jnp.full((L,), k, _i32)
                u = plsc.load_gather(bu, [rows, kk])
                a = plsc.load_gather(ba, [rows, kk])
                p = plsc.load_gather(bp, [rows, kk])
                acc_a = acc_a + u * a
                acc_p = acc_p + u * p
            oag_v[c, pl.ds(g * L, L)] = acc_a
            ope_v[c, pl.ds(g * L, L)] = acc_p
            return carry

        lax.fori_loop(0, 8, grp_body, 0)

    pltpu.sync_copy(oag_v, oag_hbm.at[wid])
    pltpu.sync_copy(ope_v, ope_hbm.at[wid])


def kernel(user_id, value_id, U, ag_V, pe_V):
    uid2 = user_id.reshape(128, 128)
    vid2 = value_id.reshape(128, 128)
    su, sa, sp = _gather_kernel(uid2, vid2, U.T, ag_V.T, pe_V.T)
    oag, ope = _dot_kernel(su, sa, sp)
    return (oag.reshape(B), ope.reshape(B))


# speedup vs baseline: 1.1521x; 1.0124x over previous
import functools

import jax
import jax.numpy as jnp
from jax import lax
from jax.experimental import pallas as pl
from jax.experimental.pallas import tpu as pltpu
from jax.experimental.pallas import tpu_sc as plsc

N = 1000000
K = 64
B = 16384

NC = 2
NS = 16
L = 16
NW = NC * NS
NBKT = (N + 127) // 128
CAP = 2 * B
SB = B + L
SENT = 0x7FFFFFFF

_mesh = plsc.VectorSubcoreMesh(core_axis_name="c", subcore_axis_name="s")
_params = pltpu.CompilerParams(needs_layout_passes=False)

_i32 = jnp.int32
_f32 = jnp.float32


@functools.partial(
    pl.kernel,
    out_type=(
        jax.ShapeDtypeStruct((SB, 128), _f32),
        jax.ShapeDtypeStruct((SB, 128), _f32),
        jax.ShapeDtypeStruct((SB, 128), _f32),
    ),
    mesh=_mesh,
    compiler_params=_params,
    scratch_types=[
        pltpu.VMEM((16, 128), _i32),
        pltpu.VMEM((CAP + L,), _i32),
        pltpu.VMEM((CAP + L,), _i32),
        pltpu.VMEM((256,), _i32),
        pltpu.VMEM((272,), _i32),
        pltpu.VMEM((256,), _i32),
        pltpu.VMEM((32,), _i32),
        pltpu.VMEM((K, 128), _f32),
        pltpu.VMEM((K, 128), _f32),
        pltpu.VMEM((K, 128), _f32),
        pltpu.VMEM((K, 128), _f32),
        pltpu.VMEM((K, 128), _f32),
        pltpu.VMEM((K, 128), _f32),
        pltpu.VMEM((L, 128), _f32),
        pltpu.VMEM((L, 128), _f32),
        pltpu.VMEM((L, 128), _f32),
        pltpu.VMEM((1, L), _i32),
        pltpu.VMEM((1, L), _i32),
        pltpu.SemaphoreType.DMA,
        pltpu.SemaphoreType.DMA,
        pltpu.SemaphoreType.DMA,
    ],
)
def _gather_kernel(uid_hbm, vid_hbm, Ut_hbm, agt_hbm, pet_hbm,
                   su_hbm, sa_hbm, sp_hbm,
                   scan_v, vals_v, svals_v, cnts_v, starts_v, offs_v, roll_v,
                   sau, saa, sap, sbu, sba, sbp,
                   fr_u, fr_a, fr_p, fp_u, fp_v,
                   semA, semB, semF):
    wid = lax.axis_index("s") * NC + lax.axis_index("c")
    lo = wid * NBKT // NW
    hi = (wid + 1) * NBKT // NW
    iota = lax.iota(_i32, L)
    ones = jnp.ones((L,), _i32)
    zeros = jnp.zeros((L,), _i32)

    cnt = jnp.asarray(0, _i32)
    for ch in range(16):
        tag = 0 if ch < 8 else 1
        src = uid_hbm if ch < 8 else vid_hbm
        row0 = (ch % 8) * 16
        pltpu.sync_copy(src.at[pl.ds(row0, 16)], scan_v)

        def scan_body(g, cnt, tag=tag, row0=row0):
            ids = scan_v[lax.shift_right_logical(g, 3),
                         pl.ds(jnp.bitwise_and(g, 7) * L, L)]
            bkt = lax.shift_right_logical(ids, 7)
            m = (bkt >= lo) & (bkt < hi)
            pos = row0 * 128 + g * L + iota
            val = (pos | (tag << 14) | (jnp.bitwise_and(ids, 127) << 15)
                   | ((bkt - lo) << 22))
            plsc.store_compressed(vals_v.at[pl.ds(cnt, L)], val, mask=m)
            return cnt + plsc.all_reduce_population_count(m)[0]

        cnt = lax.fori_loop(0, 128, scan_body, cnt)

    ngrp = lax.div(cnt + (L - 1), jnp.asarray(L, _i32))

    for j in range(16):
        cnts_v[pl.ds(j * L, L)] = zeros

    def count_body(g, carry):
        v16 = vals_v[pl.ds(g * L, L)]
        valid = (g * L + iota) < cnt
        br = jnp.bitwise_and(lax.shift_right_logical(v16, 22), 255)
        plsc.addupdate_scatter(cnts_v, [br], ones, mask=valid)
        return carry

    lax.fori_loop(0, ngrp, count_body, 0)

    carry = jnp.asarray(0, _i32)
    for j in range(16):
        c16 = cnts_v[pl.ds(j * L, L)]
        inc = plsc.cumsum(c16)
        starts_v[pl.ds(j * L, L)] = inc - c16 + carry
        offs_v[pl.ds(j * L, L)] = inc - c16 + carry
        carry = carry + jnp.sum(c16)
    starts_v[pl.ds(256, L)] = jnp.full((L,), carry, _i32)

    def sort_body(g, carry):
        v16 = vals_v[pl.ds(g * L, L)]
        valid = (g * L + iota) < cnt
        key = jnp.where(valid, v16, SENT)
        sk, sv = plsc.sort_key_val(key, v16)
        vs = sk != SENT
        br = jnp.bitwise_and(lax.shift_right_logical(sk, 22), 255)
        roll_v[pl.ds(0, L)] = jnp.full((L,), -1, _i32)
        roll_v[pl.ds(1, L)] = br
        seg = br != roll_v[pl.ds(0, L)]
        rank = iota - plsc.cummax(jnp.where(seg, iota, 0))
        base = plsc.load_gather(offs_v, [br])
        plsc.store_scatter(svals_v, [base + rank], sv, mask=vs)
        plsc.addupdate_scatter(offs_v, [br], ones, mask=vs)
        return carry

    lax.fori_loop(0, ngrp, sort_body, 0)

    def issue(b, slabs, sem):
        bb = pl.multiple_of(b * 128, 128)
        pltpu.async_copy(Ut_hbm.at[:, pl.ds(bb, 128)], slabs[0], sem)
        pltpu.async_copy(agt_hbm.at[:, pl.ds(bb, 128)], slabs[1], sem)
        pltpu.async_copy(pet_hbm.at[:, pl.ds(bb, 128)], slabs[2], sem)

    def wait_slabs(slabs, sem):
        for s in slabs:
            pltpu.make_async_copy(Ut_hbm.at[:, pl.ds(0, 128)], s, sem).wait()

    def drain_flush():
        pltpu.make_async_copy(fr_u, su_hbm.at[pl.ds(0, L)], semF).wait()
        pltpu.make_async_copy(fr_a, sa_hbm.at[pl.ds(0, L)], semF).wait()
        pltpu.make_async_copy(fr_p, sp_hbm.at[pl.ds(0, L)], semF).wait()

    def extract(slabs, off, valid_n, out_flag):
        v16 = svals_v[pl.ds(off, L)]
        valid = iota < valid_n
        pos = jnp.bitwise_and(v16, 16383)
        tagv = jnp.bitwise_and(lax.shift_right_logical(v16, 14), 1)
        col = jnp.bitwise_and(lax.shift_right_logical(v16, 15), 127)
        mu = valid & (tagv == 0)
        mv = valid & (tagv == 1)
        ru = plsc.cumsum(jnp.where(mu, 1, 0)) - 1
        rv = plsc.cumsum(jnp.where(mv, 1, 0)) - 1

        @pl.when(out_flag == 1)
        def _():
            drain_flush()

        fp_u[0, pl.ds(0, L)] = B + iota
        fp_v[0, pl.ds(0, L)] = B + iota
        plsc.store_scatter(fp_u, [zeros, ru], pos, mask=mu)
        plsc.store_scatter(fp_v, [zeros, rv], pos, mask=mv)
        for k in range(K):
            kk = jnp.full((L,), k, _i32)
            gu = plsc.load_gather(slabs[0], [kk, col])
            plsc.store_scatter(fr_u, [ru, kk], gu, mask=mu)
            ga = plsc.load_gather(slabs[1], [kk, col])
            plsc.store_scatter(fr_a, [rv, kk], ga, mask=mv)
            gp = plsc.load_gather(slabs[2], [kk, col])
            plsc.store_scatter(fr_p, [rv, kk], gp, mask=mv)
        pltpu.async_copy(fr_u, su_hbm.at[fp_u.at[0]], semF)
        pltpu.async_copy(fr_a, sa_hbm.at[fp_v.at[0]], semF)
        pltpu.async_copy(fr_p, sp_hbm.at[fp_v.at[0]], semF)

    def process_bucket(b, slabs, out_flag):
        bounds = starts_v[pl.ds(b - lo, L)]
        s0 = bounds[0]
        nloc = bounds[1] - s0

        def egroup(j, oflag):
            extract(slabs, s0 + j * L, jnp.minimum(nloc - j * L, L), oflag)
            return jnp.asarray(1, _i32)

        negrp = lax.div(nloc + (L - 1), jnp.asarray(L, _i32))
        return lax.fori_loop(0, negrp, egroup, out_flag)

    slabsA = (sau, saa, sap)
    slabsB = (sbu, sba, sbp)

    @pl.when(lo < hi)
    def _():
        issue(lo, slabsA, semA)

    @pl.when(lo + 1 < hi)
    def _():
        issue(lo + 1, slabsB, semB)

    def pair_body(i, out_flag):
        b0 = lo + 2 * i
        b1 = b0 + 1

        wait_slabs(slabsA, semA)
        out_flag = process_bucket(b0, slabsA, out_flag)

        @pl.when(b0 + 2 < hi)
        def _():
            issue(b0 + 2, slabsA, semA)

        @pl.when(b1 < hi)
        def _():
            wait_slabs(slabsB, semB)

        out_flag = process_bucket(b1, slabsB, out_flag)

        @pl.when(b1 + 2 < hi)
        def _():
            issue(b1 + 2, slabsB, semB)

        return out_flag

    npairs = lax.div(hi - lo + 1, jnp.asarray(2, _i32))
    out_flag = lax.fori_loop(0, npairs, pair_body, jnp.asarray(0, _i32))

    @pl.when(out_flag == 1)
    def _():
        drain_flush()


@functools.partial(
    pl.kernel,
    out_type=(
        jax.ShapeDtypeStruct((NW, 4, 128), _f32),
        jax.ShapeDtypeStruct((NW, 4, 128), _f32),
    ),
    mesh=_mesh,
    compiler_params=_params,
    scratch_types=[
        pltpu.VMEM((128, 128), _f32),
        pltpu.VMEM((128, 128), _f32),
        pltpu.VMEM((128, 128), _f32),
        pltpu.VMEM((4, 128), _f32),
        pltpu.VMEM((4, 128), _f32),
        pltpu.SemaphoreType.DMA,
    ],
)
def _dot_kernel(su_hbm, sa_hbm, sp_hbm, oag_hbm, ope_hbm,
                bu, ba, bp, oag_v, ope_v, sem):
    wid = lax.axis_index("s") * NC + lax.axis_index("c")
    base = wid * 512
    iota = lax.iota(_i32, L)
    for c in range(4):
        pltpu.async_copy(su_hbm.at[pl.ds(base + c * 128, 128)], bu, sem)
        pltpu.async_copy(sa_hbm.at[pl.ds(base + c * 128, 128)], ba, sem)
        pltpu.async_copy(sp_hbm.at[pl.ds(base + c * 128, 128)], bp, sem)
        pltpu.make_async_copy(su_hbm.at[pl.ds(0, 128)], bu, sem).wait()
        pltpu.make_async_copy(sa_hbm.at[pl.ds(0, 128)], ba, sem).wait()
        pltpu.make_async_copy(sp_hbm.at[pl.ds(0, 128)], bp, sem).wait()

        def grp_body(g, carry, c=c):
            rows = iota + g * L
            acc_a = jnp.zeros((L,), _f32)
            acc_p = jnp.zeros((L,), _f32)
            for k in range(K):
                kk = jnp.full((L,), k, _i32)
                u = plsc.load_gather(bu, [rows, kk])
                a = plsc.load_gather(ba, [rows, kk])
                p = plsc.load_gather(bp, [rows, kk])
                acc_a = acc_a + u * a
                acc_p = acc_p + u * p
            oag_v[c, pl.ds(g * L, L)] = acc_a
            ope_v[c, pl.ds(g * L, L)] = acc_p
            return carry

        lax.fori_loop(0, 8, grp_body, 0)

    pltpu.sync_copy(oag_v, oag_hbm.at[wid])
    pltpu.sync_copy(ope_v, ope_hbm.at[wid])


def kernel(user_id, value_id, U, ag_V, pe_V):
    uid2 = user_id.reshape(128, 128)
    vid2 = value_id.reshape(128, 128)
    su, sa, sp = _gather_kernel(uid2, vid2, U.T, ag_V.T, pe_V.T)
    oag, ope = _dot_kernel(su, sa, sp)
    return (oag.reshape(B), ope.reshape(B))


# speedup vs baseline: 3.0377x; 2.6367x over previous
import functools

import jax
import jax.numpy as jnp
from jax import lax
from jax.experimental import pallas as pl
from jax.experimental.pallas import tpu as pltpu
from jax.experimental.pallas import tpu_sc as plsc

N = 1000000
K = 64
B = 16384

NC = 2
NS = 16
L = 16
NW = NC * NS
NBKT = (N + 127) // 128
CAP = 2 * B
SB = B + L
SENT = 0x7FFFFFFF

_mesh = plsc.VectorSubcoreMesh(core_axis_name="c", subcore_axis_name="s")
_params = pltpu.CompilerParams(needs_layout_passes=False)

_i32 = jnp.int32
_f32 = jnp.float32


@functools.partial(
    pl.kernel,
    out_type=(
        jax.ShapeDtypeStruct((SB, 128), _f32),
        jax.ShapeDtypeStruct((SB, 128), _f32),
        jax.ShapeDtypeStruct((SB, 128), _f32),
    ),
    mesh=_mesh,
    compiler_params=_params,
    scratch_types=[
        pltpu.VMEM((16, 128), _i32),
        pltpu.VMEM((CAP + L,), _i32),
        pltpu.VMEM((256,), _i32),
        pltpu.VMEM((272,), _i32),
        pltpu.VMEM((256,), _i32),
        pltpu.VMEM((32,), _i32),
        pltpu.VMEM((K, 128), _f32),
        pltpu.VMEM((K, 128), _f32),
        pltpu.VMEM((K, 128), _f32),
        pltpu.VMEM((K, 128), _f32),
        pltpu.VMEM((K, 128), _f32),
        pltpu.VMEM((K, 128), _f32),
        pltpu.VMEM((64, 128), _f32),
        pltpu.VMEM((64, 128), _f32),
        pltpu.VMEM((64, 128), _f32),
        pltpu.VMEM((1, 64), _i32),
        pltpu.VMEM((1, 64), _i32),
        pltpu.SemaphoreType.DMA,
        pltpu.SemaphoreType.DMA,
        pltpu.SemaphoreType.DMA,
    ],
)
def _gather_kernel(uid_hbm, vid_hbm, Ut_hbm, agt_hbm, pet_hbm,
                   su_hbm, sa_hbm, sp_hbm,
                   scan_v, svals_v, cnts_v, starts_v, offs_v, roll_v,
                   sau, saa, sap, sbu, sba, sbp,
                   fr_u, fr_a, fr_p, fp_u, fp_v,
                   semA, semB, semF):
    wid = lax.axis_index("s") * NC + lax.axis_index("c")
    lo = wid * NBKT // NW
    hi = (wid + 1) * NBKT // NW
    iota = lax.iota(_i32, L)
    ones = jnp.ones((L,), _i32)
    zeros = jnp.zeros((L,), _i32)

    for j in range(16):
        cnts_v[pl.ds(j * L, L)] = zeros

    for ch in range(16):
        src = uid_hbm if ch < 8 else vid_hbm
        row0 = (ch % 8) * 16
        pltpu.sync_copy(src.at[pl.ds(row0, 16)], scan_v)

        def cnt_body(g, carry):
            ids = scan_v[lax.shift_right_logical(g, 3),
                         pl.ds(jnp.bitwise_and(g, 7) * L, L)]
            bkt = lax.shift_right_logical(ids, 7)
            m = (bkt >= lo) & (bkt < hi)
            br = jnp.bitwise_and(bkt - lo, 255)
            plsc.addupdate_scatter(cnts_v, [br], ones, mask=m)
            return carry

        lax.fori_loop(0, 128, cnt_body, 0)

    carry = jnp.asarray(0, _i32)
    for j in range(16):
        c16 = cnts_v[pl.ds(j * L, L)]
        inc = plsc.cumsum(c16)
        starts_v[pl.ds(j * L, L)] = inc - c16 + carry
        offs_v[pl.ds(j * L, L)] = inc - c16 + carry
        carry = carry + jnp.sum(c16)
    starts_v[pl.ds(256, L)] = jnp.full((L,), carry, _i32)

    for ch in range(16):
        tag = 0 if ch < 8 else 1
        src = uid_hbm if ch < 8 else vid_hbm
        row0 = (ch % 8) * 16
        pltpu.sync_copy(src.at[pl.ds(row0, 16)], scan_v)

        def sort_body(g, carry, tag=tag, row0=row0):
            ids = scan_v[lax.shift_right_logical(g, 3),
                         pl.ds(jnp.bitwise_and(g, 7) * L, L)]
            bkt = lax.shift_right_logical(ids, 7)
            m = (bkt >= lo) & (bkt < hi)
            pos = row0 * 128 + g * L + iota
            val = pos | (tag << 14) | (jnp.bitwise_and(ids, 127) << 15)
            key = jnp.where(m, jnp.bitwise_and(bkt - lo, 255), 256)
            sk, sv = plsc.sort_key_val(key, val)
            vs = sk != 256
            br = jnp.bitwise_and(sk, 255)
            roll_v[pl.ds(0, L)] = jnp.full((L,), -1, _i32)
            roll_v[pl.ds(1, L)] = br
            seg = br != roll_v[pl.ds(0, L)]
            rank = iota - plsc.cummax(jnp.where(seg, iota, 0))
            base = plsc.load_gather(offs_v, [br])
            plsc.store_scatter(svals_v, [base + rank], sv, mask=vs)
            plsc.addupdate_scatter(offs_v, [br], ones, mask=vs)
            return carry

        lax.fori_loop(0, 128, sort_body, 0)

    def issue(b, slabs, sem):
        bb = pl.multiple_of(b * 128, 128)
        pltpu.async_copy(Ut_hbm.at[:, pl.ds(bb, 128)], slabs[0], sem)
        pltpu.async_copy(agt_hbm.at[:, pl.ds(bb, 128)], slabs[1], sem)
        pltpu.async_copy(pet_hbm.at[:, pl.ds(bb, 128)], slabs[2], sem)

    def wait_slabs(slabs, sem):
        for s in slabs:
            pltpu.make_async_copy(Ut_hbm.at[:, pl.ds(0, 128)], s, sem).wait()

    def reinit_fp(fp):
        for j in range(4):
            fp[0, pl.ds(j * L, L)] = B + iota

    def flush_u():
        pltpu.async_copy(fr_u, su_hbm.at[fp_u.at[0]], semF)
        pltpu.make_async_copy(fr_u, su_hbm.at[pl.ds(0, 64)], semF).wait()
        reinit_fp(fp_u)

    def flush_v():
        pltpu.async_copy(fr_a, sa_hbm.at[fp_v.at[0]], semF)
        pltpu.async_copy(fr_p, sp_hbm.at[fp_v.at[0]], semF)
        pltpu.make_async_copy(fr_a, sa_hbm.at[pl.ds(0, 64)], semF).wait()
        pltpu.make_async_copy(fr_p, sp_hbm.at[pl.ds(0, 64)], semF).wait()
        reinit_fp(fp_v)

    def extract(slabs, off, valid_n, ufill, vfill):
        v16 = svals_v[pl.ds(off, L)]
        valid = iota < valid_n
        pos = jnp.bitwise_and(v16, 16383)
        tagv = jnp.bitwise_and(lax.shift_right_logical(v16, 14), 1)
        col = jnp.bitwise_and(lax.shift_right_logical(v16, 15), 127)
        mu = valid & (tagv == 0)
        mv = valid & (tagv == 1)
        ru = plsc.cumsum(jnp.where(mu, 1, 0)) - 1 + ufill
        rv = plsc.cumsum(jnp.where(mv, 1, 0)) - 1 + vfill
        plsc.store_scatter(fp_u, [zeros, ru], pos, mask=mu)
        plsc.store_scatter(fp_v, [zeros, rv], pos, mask=mv)
        for k in range(K):
            kk = jnp.full((L,), k, _i32)
            gu = plsc.load_gather(slabs[0], [kk, col])
            plsc.store_scatter(fr_u, [ru, kk], gu, mask=mu)
            ga = plsc.load_gather(slabs[1], [kk, col])
            plsc.store_scatter(fr_a, [rv, kk], ga, mask=mv)
            gp = plsc.load_gather(slabs[2], [kk, col])
            plsc.store_scatter(fr_p, [rv, kk], gp, mask=mv)
        nu = plsc.all_reduce_population_count(mu)[0]
        nv = plsc.all_reduce_population_count(mv)[0]
        return ufill + nu, vfill + nv

    def process_bucket(b, slabs, fills):
        bounds = starts_v[pl.ds(b - lo, L)]
        s0 = bounds[0]
        nloc = bounds[1] - s0

        def egroup(j, fills):
            ufill, vfill = fills
            ufill, vfill = extract(
                slabs, s0 + j * L, jnp.minimum(nloc - j * L, L), ufill, vfill)
            fu = ufill > 48

            @pl.when(fu)
            def _():
                flush_u()

            ufill = jnp.where(fu, 0, ufill)
            fv = vfill > 48

            @pl.when(fv)
            def _():
                flush_v()

            vfill = jnp.where(fv, 0, vfill)
            return (ufill, vfill)

        negrp = lax.div(nloc + (L - 1), jnp.asarray(L, _i32))
        return lax.fori_loop(0, negrp, egroup, fills)

    slabsA = (sau, saa, sap)
    slabsB = (sbu, sba, sbp)

    reinit_fp(fp_u)
    reinit_fp(fp_v)

    @pl.when(lo < hi)
    def _():
        issue(lo, slabsA, semA)

    @pl.when(lo + 1 < hi)
    def _():
        issue(lo + 1, slabsB, semB)

    def pair_body(i, fills):
        b0 = lo + 2 * i
        b1 = b0 + 1

        wait_slabs(slabsA, semA)
        fills = process_bucket(b0, slabsA, fills)

        @pl.when(b0 + 2 < hi)
        def _():
            issue(b0 + 2, slabsA, semA)

        @pl.when(b1 < hi)
        def _():
            wait_slabs(slabsB, semB)

        fills = process_bucket(b1, slabsB, fills)

        @pl.when(b1 + 2 < hi)
        def _():
            issue(b1 + 2, slabsB, semB)

        return fills

    npairs = lax.div(hi - lo + 1, jnp.asarray(2, _i32))
    z = jnp.asarray(0, _i32)
    ufill, vfill = lax.fori_loop(0, npairs, pair_body, (z, z))

    @pl.when(ufill > 0)
    def _():
        flush_u()

    @pl.when(vfill > 0)
    def _():
        flush_v()


@functools.partial(
    pl.kernel,
    out_type=(
        jax.ShapeDtypeStruct((NW, 4, 128), _f32),
        jax.ShapeDtypeStruct((NW, 4, 128), _f32),
    ),
    mesh=_mesh,
    compiler_params=_params,
    scratch_types=[
        pltpu.VMEM((128, 128), _f32),
        pltpu.VMEM((128, 128), _f32),
        pltpu.VMEM((128, 128), _f32),
        pltpu.VMEM((4, 128), _f32),
        pltpu.VMEM((4, 128), _f32),
        pltpu.SemaphoreType.DMA,
    ],
)
def _dot_kernel(su_hbm, sa_hbm, sp_hbm, oag_hbm, ope_hbm,
                bu, ba, bp, oag_v, ope_v, sem):
    wid = lax.axis_index("s") * NC + lax.axis_index("c")
    base = wid * 512
    iota = lax.iota(_i32, L)
    for c in range(4):
        pltpu.async_copy(su_hbm.at[pl.ds(base + c * 128, 128)], bu, sem)
        pltpu.async_copy(sa_hbm.at[pl.ds(base + c * 128, 128)], ba, sem)
        pltpu.async_copy(sp_hbm.at[pl.ds(base + c * 128, 128)], bp, sem)
        pltpu.make_async_copy(su_hbm.at[pl.ds(0, 128)], bu, sem).wait()
        pltpu.make_async_copy(sa_hbm.at[pl.ds(0, 128)], ba, sem).wait()
        pltpu.make_async_copy(sp_hbm.at[pl.ds(0, 128)], bp, sem).wait()

        def grp_body(g, carry, c=c):
            rows = iota + g * L
            acc_a = jnp.zeros((L,), _f32)
            acc_p = jnp.zeros((L,), _f32)
            for k in range(K):
                kk = jnp.full((L,), k, _i32)
                u = plsc.load_gather(bu, [rows, kk])
                a = plsc.load_gather(ba, [rows, kk])
                p = plsc.load_gather(bp, [rows, kk])
                acc_a = acc_a + u * a
                acc_p = acc_p + u * p
            oag_v[c, pl.ds(g * L, L)] = acc_a
            ope_v[c, pl.ds(g * L, L)] = acc_p
            return carry

        lax.fori_loop(0, 8, grp_body, 0)

    pltpu.sync_copy(oag_v, oag_hbm.at[wid])
    pltpu.sync_copy(ope_v, ope_hbm.at[wid])


def kernel(user_id, value_id, U, ag_V, pe_V):
    uid2 = user_id.reshape(128, 128)
    vid2 = value_id.reshape(128, 128)
    su, sa, sp = _gather_kernel(uid2, vid2, U.T, ag_V.T, pe_V.T)
    oag, ope = _dot_kernel(su, sa, sp)
    return (oag.reshape(B), ope.reshape(B))


# speedup vs baseline: 3.1020x; 1.0212x over previous
import functools

import jax
import jax.numpy as jnp
from jax import lax
from jax.experimental import pallas as pl
from jax.experimental.pallas import tpu as pltpu
from jax.experimental.pallas import tpu_sc as plsc

N = 1000000
K = 64
B = 16384

NC = 2
NS = 16
L = 16
NW = NC * NS
NBKT = (N + 127) // 128
CAP = 2 * B
SB = B + L
SENT = 0x7FFFFFFF

_mesh = plsc.VectorSubcoreMesh(core_axis_name="c", subcore_axis_name="s")
_params = pltpu.CompilerParams(needs_layout_passes=False)

_i32 = jnp.int32
_f32 = jnp.float32


@functools.partial(
    pl.kernel,
    out_type=(
        jax.ShapeDtypeStruct((SB, 128), _f32),
        jax.ShapeDtypeStruct((SB, 128), _f32),
        jax.ShapeDtypeStruct((SB, 128), _f32),
    ),
    mesh=_mesh,
    compiler_params=_params,
    scratch_types=[
        pltpu.VMEM((16, 128), _i32),
        pltpu.VMEM((CAP + L,), _i32),
        pltpu.VMEM((272,), _i32),
        pltpu.VMEM((272,), _i32),
        pltpu.VMEM((272,), _i32),
        pltpu.VMEM((256,), _i32),
        pltpu.VMEM((32,), _i32),
        pltpu.VMEM((K, 128), _f32),
        pltpu.VMEM((K, 128), _f32),
        pltpu.VMEM((K, 128), _f32),
        pltpu.VMEM((K, 128), _f32),
        pltpu.VMEM((K, 128), _f32),
        pltpu.VMEM((K, 128), _f32),
        pltpu.VMEM((64, 128), _f32),
        pltpu.VMEM((64, 128), _f32),
        pltpu.VMEM((64, 128), _f32),
        pltpu.VMEM((1, 64), _i32),
        pltpu.VMEM((1, 64), _i32),
        pltpu.SemaphoreType.DMA,
        pltpu.SemaphoreType.DMA,
        pltpu.SemaphoreType.DMA,
    ],
)
def _gather_kernel(uid_hbm, vid_hbm, Ut_hbm, agt_hbm, pet_hbm,
                   su_hbm, sa_hbm, sp_hbm,
                   scan_v, svals_v, ucnt_v, vcnt_v, starts_v, offs_v, roll_v,
                   sau, saa, sap, sbu, sba, sbp,
                   fr_u, fr_a, fr_p, fp_u, fp_v,
                   semA, semB, semF):
    wid = lax.axis_index("s") * NC + lax.axis_index("c")
    lo = wid * NBKT // NW
    hi = (wid + 1) * NBKT // NW
    iota = lax.iota(_i32, L)
    ones = jnp.ones((L,), _i32)
    zeros = jnp.zeros((L,), _i32)

    for j in range(17):
        ucnt_v[pl.ds(j * L, L)] = zeros
        vcnt_v[pl.ds(j * L, L)] = zeros

    for ch in range(16):
        src = uid_hbm if ch < 8 else vid_hbm
        dst_cnt = ucnt_v if ch < 8 else vcnt_v
        row0 = (ch % 8) * 16
        pltpu.sync_copy(src.at[pl.ds(row0, 16)], scan_v)

        def cnt_body(g, carry, dst_cnt=dst_cnt):
            ids = scan_v[lax.shift_right_logical(g, 3),
                         pl.ds(jnp.bitwise_and(g, 7) * L, L)]
            bkt = lax.shift_right_logical(ids, 7)
            m = (bkt >= lo) & (bkt < hi)
            br = jnp.bitwise_and(bkt - lo, 255)
            plsc.addupdate_scatter(dst_cnt, [br], ones, mask=m)
            return carry

        lax.fori_loop(0, 128, cnt_body, 0)

    carry = jnp.asarray(0, _i32)
    for j in range(16):
        c16 = ucnt_v[pl.ds(j * L, L)] + vcnt_v[pl.ds(j * L, L)]
        inc = plsc.cumsum(c16)
        starts_v[pl.ds(j * L, L)] = inc - c16 + carry
        offs_v[pl.ds(j * L, L)] = inc - c16 + carry
        carry = carry + jnp.sum(c16)
    starts_v[pl.ds(256, L)] = jnp.full((L,), carry, _i32)

    for ch in range(16):
        tag = 0 if ch < 8 else 1
        src = uid_hbm if ch < 8 else vid_hbm
        row0 = (ch % 8) * 16
        pltpu.sync_copy(src.at[pl.ds(row0, 16)], scan_v)

        def sort_body(g, carry, tag=tag, row0=row0):
            ids = scan_v[lax.shift_right_logical(g, 3),
                         pl.ds(jnp.bitwise_and(g, 7) * L, L)]
            bkt = lax.shift_right_logical(ids, 7)
            m = (bkt >= lo) & (bkt < hi)
            pos = row0 * 128 + g * L + iota
            val = pos | (tag << 14) | (jnp.bitwise_and(ids, 127) << 15)
            key = jnp.where(m, jnp.bitwise_and(bkt - lo, 255), 256)
            sk, sv = plsc.sort_key_val(key, val)
            vs = sk != 256
            br = jnp.bitwise_and(sk, 255)
            roll_v[pl.ds(0, L)] = jnp.full((L,), -1, _i32)
            roll_v[pl.ds(1, L)] = br
            seg = br != roll_v[pl.ds(0, L)]
            rank = iota - plsc.cummax(jnp.where(seg, iota, 0))
            base = plsc.load_gather(offs_v, [br])
            plsc.store_scatter(svals_v, [base + rank], sv, mask=vs)
            plsc.addupdate_scatter(offs_v, [br], ones, mask=vs)
            return carry

        lax.fori_loop(0, 128, sort_body, 0)

    def bucket_flags(b):
        rel = jnp.bitwise_and(b - lo, 255)
        hasu = ucnt_v[pl.ds(rel, L)][0] > 0
        hasv = vcnt_v[pl.ds(rel, L)][0] > 0
        return hasu, hasv

    def issue(b, slabs, sem):
        bb = pl.multiple_of(b * 128, 128)
        hasu, hasv = bucket_flags(b)

        @pl.when(hasu)
        def _():
            pltpu.async_copy(Ut_hbm.at[:, pl.ds(bb, 128)], slabs[0], sem)

        @pl.when(hasv)
        def _():
            pltpu.async_copy(agt_hbm.at[:, pl.ds(bb, 128)], slabs[1], sem)
            pltpu.async_copy(pet_hbm.at[:, pl.ds(bb, 128)], slabs[2], sem)

    def wait_slabs(b, slabs, sem):
        hasu, hasv = bucket_flags(b)

        @pl.when(hasu)
        def _():
            pltpu.make_async_copy(
                Ut_hbm.at[:, pl.ds(0, 128)], slabs[0], sem).wait()

        @pl.when(hasv)
        def _():
            pltpu.make_async_copy(
                Ut_hbm.at[:, pl.ds(0, 128)], slabs[1], sem).wait()
            pltpu.make_async_copy(
                Ut_hbm.at[:, pl.ds(0, 128)], slabs[2], sem).wait()

    def reinit_fp(fp):
        for j in range(4):
            fp[0, pl.ds(j * L, L)] = B + iota

    def flush_u():
        pltpu.async_copy(fr_u, su_hbm.at[fp_u.at[0]], semF)
        pltpu.make_async_copy(fr_u, su_hbm.at[pl.ds(0, 64)], semF).wait()
        reinit_fp(fp_u)

    def flush_v():
        pltpu.async_copy(fr_a, sa_hbm.at[fp_v.at[0]], semF)
        pltpu.async_copy(fr_p, sp_hbm.at[fp_v.at[0]], semF)
        pltpu.make_async_copy(fr_a, sa_hbm.at[pl.ds(0, 64)], semF).wait()
        pltpu.make_async_copy(fr_p, sp_hbm.at[pl.ds(0, 64)], semF).wait()
        reinit_fp(fp_v)

    def extract(slabs, off, valid_n, ufill, vfill):
        v16 = svals_v[pl.ds(off, L)]
        valid = iota < valid_n
        pos = jnp.bitwise_and(v16, 16383)
        tagv = jnp.bitwise_and(lax.shift_right_logical(v16, 14), 1)
        col = jnp.bitwise_and(lax.shift_right_logical(v16, 15), 127)
        mu = valid & (tagv == 0)
        mv = valid & (tagv == 1)
        ru = plsc.cumsum(jnp.where(mu, 1, 0)) - 1 + ufill
        rv = plsc.cumsum(jnp.where(mv, 1, 0)) - 1 + vfill
        plsc.store_scatter(fp_u, [zeros, ru], pos, mask=mu)
        plsc.store_scatter(fp_v, [zeros, rv], pos, mask=mv)
        for k in range(K):
            kk = jnp.full((L,), k, _i32)
            gu = plsc.load_gather(slabs[0], [kk, col])
            plsc.store_scatter(fr_u, [ru, kk], gu, mask=mu)
            ga = plsc.load_gather(slabs[1], [kk, col])
            plsc.store_scatter(fr_a, [rv, kk], ga, mask=mv)
            gp = plsc.load_gather(slabs[2], [kk, col])
            plsc.store_scatter(fr_p, [rv, kk], gp, mask=mv)
        nu = plsc.all_reduce_population_count(mu)[0]
        nv = plsc.all_reduce_population_count(mv)[0]
        return ufill + nu, vfill + nv

    def process_bucket(b, slabs, fills):
        bounds = starts_v[pl.ds(b - lo, L)]
        s0 = bounds[0]
        nloc = bounds[1] - s0

        def egroup(j, fills):
            ufill, vfill = fills
            ufill, vfill = extract(
                slabs, s0 + j * L, jnp.minimum(nloc - j * L, L), ufill, vfill)
            fu = ufill > 48

            @pl.when(fu)
            def _():
                flush_u()

            ufill = jnp.where(fu, 0, ufill)
            fv = vfill > 48

            @pl.when(fv)
            def _():
                flush_v()

            vfill = jnp.where(fv, 0, vfill)
            return (ufill, vfill)

        negrp = lax.div(nloc + (L - 1), jnp.asarray(L, _i32))
        return lax.fori_loop(0, negrp, egroup, fills)

    slabsA = (sau, saa, sap)
    slabsB = (sbu, sba, sbp)

    reinit_fp(fp_u)
    reinit_fp(fp_v)

    @pl.when(lo < hi)
    def _():
        issue(lo, slabsA, semA)

    @pl.when(lo + 1 < hi)
    def _():
        issue(lo + 1, slabsB, semB)

    def pair_body(i, fills):
        b0 = lo + 2 * i
        b1 = b0 + 1

        wait_slabs(b0, slabsA, semA)
        fills = process_bucket(b0, slabsA, fills)

        @pl.when(b0 + 2 < hi)
        def _():
            issue(b0 + 2, slabsA, semA)

        @pl.when(b1 < hi)
        def _():
            wait_slabs(b1, slabsB, semB)

        fills = process_bucket(b1, slabsB, fills)

        @pl.when(b1 + 2 < hi)
        def _():
            issue(b1 + 2, slabsB, semB)

        return fills

    npairs = lax.div(hi - lo + 1, jnp.asarray(2, _i32))
    z = jnp.asarray(0, _i32)
    ufill, vfill = lax.fori_loop(0, npairs, pair_body, (z, z))

    @pl.when(ufill > 0)
    def _():
        flush_u()

    @pl.when(vfill > 0)
    def _():
        flush_v()


@functools.partial(
    pl.kernel,
    out_type=(
        jax.ShapeDtypeStruct((NW, 4, 128), _f32),
        jax.ShapeDtypeStruct((NW, 4, 128), _f32),
    ),
    mesh=_mesh,
    compiler_params=_params,
    scratch_types=[
        pltpu.VMEM((128, 128), _f32),
        pltpu.VMEM((128, 128), _f32),
        pltpu.VMEM((128, 128), _f32),
        pltpu.VMEM((4, 128), _f32),
        pltpu.VMEM((4, 128), _f32),
        pltpu.SemaphoreType.DMA,
    ],
)
def _dot_kernel(su_hbm, sa_hbm, sp_hbm, oag_hbm, ope_hbm,
                bu, ba, bp, oag_v, ope_v, sem):
    wid = lax.axis_index("s") * NC + lax.axis_index("c")
    base = wid * 512
    iota = lax.iota(_i32, L)
    for c in range(4):
        pltpu.async_copy(su_hbm.at[pl.ds(base + c * 128, 128)], bu, sem)
        pltpu.async_copy(sa_hbm.at[pl.ds(base + c * 128, 128)], ba, sem)
        pltpu.async_copy(sp_hbm.at[pl.ds(base + c * 128, 128)], bp, sem)
        pltpu.make_async_copy(su_hbm.at[pl.ds(0, 128)], bu, sem).wait()
        pltpu.make_async_copy(sa_hbm.at[pl.ds(0, 128)], ba, sem).wait()
        pltpu.make_async_copy(sp_hbm.at[pl.ds(0, 128)], bp, sem).wait()

        def grp_body(g, carry, c=c):
            rows = iota + g * L
            acc_a = jnp.zeros((L,), _f32)
            acc_p = jnp.zeros((L,), _f32)
            for k in range(K):
                kk = jnp.full((L,), k, _i32)
                u = plsc.load_gather(bu, [rows, kk])
                a = plsc.load_gather(ba, [rows, kk])
                p = plsc.load_gather(bp, [rows, kk])
                acc_a = acc_a + u * a
                acc_p = acc_p + u * p
            oag_v[c, pl.ds(g * L, L)] = acc_a
            ope_v[c, pl.ds(g * L, L)] = acc_p
            return carry

        lax.fori_loop(0, 8, grp_body, 0)

    pltpu.sync_copy(oag_v, oag_hbm.at[wid])
    pltpu.sync_copy(ope_v, ope_hbm.at[wid])


def kernel(user_id, value_id, U, ag_V, pe_V):
    uid2 = user_id.reshape(128, 128)
    vid2 = value_id.reshape(128, 128)
    su, sa, sp = _gather_kernel(uid2, vid2, U.T, ag_V.T, pe_V.T)
    oag, ope = _dot_kernel(su, sa, sp)
    return (oag.reshape(B), ope.reshape(B))


# speedup vs baseline: 3.1305x; 1.0092x over previous
import functools

import jax
import jax.numpy as jnp
from jax import lax
from jax.experimental import pallas as pl
from jax.experimental.pallas import tpu as pltpu
from jax.experimental.pallas import tpu_sc as plsc

N = 1000000
K = 64
B = 16384

NC = 2
NS = 16
L = 16
NW = NC * NS
NBKT = (N + 127) // 128
CAP = 2 * B
SB = B + L
SENT = 0x7FFFFFFF

_mesh = plsc.VectorSubcoreMesh(core_axis_name="c", subcore_axis_name="s")
_params = pltpu.CompilerParams(needs_layout_passes=False)

_i32 = jnp.int32
_f32 = jnp.float32


@functools.partial(
    pl.kernel,
    out_type=(
        jax.ShapeDtypeStruct((SB, 128), _f32),
        jax.ShapeDtypeStruct((SB, 128), _f32),
        jax.ShapeDtypeStruct((SB, 128), _f32),
    ),
    mesh=_mesh,
    compiler_params=_params,
    scratch_types=[
        pltpu.VMEM((16, 128), _i32),
        pltpu.VMEM((CAP + L,), _i32),
        pltpu.VMEM((272,), _i32),
        pltpu.VMEM((272,), _i32),
        pltpu.VMEM((272,), _i32),
        pltpu.VMEM((256,), _i32),
        pltpu.VMEM((32,), _i32),
        pltpu.VMEM((K, 128), _f32),
        pltpu.VMEM((K, 128), _f32),
        pltpu.VMEM((K, 128), _f32),
        pltpu.VMEM((K, 128), _f32),
        pltpu.VMEM((K, 128), _f32),
        pltpu.VMEM((K, 128), _f32),
        pltpu.VMEM((64, 128), _f32),
        pltpu.VMEM((64, 128), _f32),
        pltpu.VMEM((64, 128), _f32),
        pltpu.VMEM((1, 64), _i32),
        pltpu.VMEM((1, 64), _i32),
        pltpu.SemaphoreType.DMA,
        pltpu.SemaphoreType.DMA,
        pltpu.SemaphoreType.DMA,
    ],
)
def _gather_kernel(uid_hbm, vid_hbm, Ut_hbm, agt_hbm, pet_hbm,
                   su_hbm, sa_hbm, sp_hbm,
                   scan_v, svals_v, ucnt_v, vcnt_v, starts_v, offs_v, roll_v,
                   sau, saa, sap, sbu, sba, sbp,
                   fr_u, fr_a, fr_p, fp_u, fp_v,
                   semA, semB, semF):
    wid = lax.axis_index("s") * NC + lax.axis_index("c")
    lo = wid * NBKT // NW
    hi = (wid + 1) * NBKT // NW
    iota = lax.iota(_i32, L)
    ones = jnp.ones((L,), _i32)
    zeros = jnp.zeros((L,), _i32)

    for j in range(17):
        ucnt_v[pl.ds(j * L, L)] = zeros
        vcnt_v[pl.ds(j * L, L)] = zeros

    for ch in range(16):
        src = uid_hbm if ch < 8 else vid_hbm
        dst_cnt = ucnt_v if ch < 8 else vcnt_v
        row0 = (ch % 8) * 16
        pltpu.sync_copy(src.at[pl.ds(row0, 16)], scan_v)

        def cnt_body(g, carry, dst_cnt=dst_cnt):
            ids = scan_v[lax.shift_right_logical(g, 3),
                         pl.ds(jnp.bitwise_and(g, 7) * L, L)]
            bkt = lax.shift_right_logical(ids, 7)
            m = (bkt >= lo) & (bkt < hi)
            br = jnp.bitwise_and(bkt - lo, 255)
            plsc.addupdate_scatter(dst_cnt, [br], ones, mask=m)
            return carry

        lax.fori_loop(0, 128, cnt_body, 0)

    carry = jnp.asarray(0, _i32)
    for j in range(16):
        c16 = ucnt_v[pl.ds(j * L, L)] + vcnt_v[pl.ds(j * L, L)]
        inc = plsc.cumsum(c16)
        starts_v[pl.ds(j * L, L)] = inc - c16 + carry
        offs_v[pl.ds(j * L, L)] = inc - c16 + carry
        carry = carry + jnp.sum(c16)
    starts_v[pl.ds(256, L)] = jnp.full((L,), carry, _i32)

    for ch in range(16):
        tag = 0 if ch < 8 else 1
        src = uid_hbm if ch < 8 else vid_hbm
        row0 = (ch % 8) * 16
        pltpu.sync_copy(src.at[pl.ds(row0, 16)], scan_v)

        def sort_body(g, carry, tag=tag, row0=row0):
            ids = scan_v[lax.shift_right_logical(g, 3),
                         pl.ds(jnp.bitwise_and(g, 7) * L, L)]
            bkt = lax.shift_right_logical(ids, 7)
            m = (bkt >= lo) & (bkt < hi)
            pos = row0 * 128 + g * L + iota
            val = pos | (tag << 14) | (jnp.bitwise_and(ids, 127) << 15)
            key = jnp.where(m, jnp.bitwise_and(bkt - lo, 255), 256)
            sk, sv = plsc.sort_key_val(key, val)
            vs = sk != 256
            br = jnp.bitwise_and(sk, 255)
            roll_v[pl.ds(0, L)] = jnp.full((L,), -1, _i32)
            roll_v[pl.ds(1, L)] = br
            seg = br != roll_v[pl.ds(0, L)]
            rank = iota - plsc.cummax(jnp.where(seg, iota, 0))
            base = plsc.load_gather(offs_v, [br])
            plsc.store_scatter(svals_v, [base + rank], sv, mask=vs)
            plsc.addupdate_scatter(offs_v, [br], ones, mask=vs)
            return carry

        lax.fori_loop(0, 128, sort_body, 0)

    def bucket_flags(b):
        rel = jnp.bitwise_and(b - lo, 255)
        hasu = ucnt_v[pl.ds(rel, L)][0] > 0
        hasv = vcnt_v[pl.ds(rel, L)][0] > 0
        return hasu, hasv

    def issue(b, slabs, sem):
        bb = pl.multiple_of(b * 128, 128)
        hasu, hasv = bucket_flags(b)

        @pl.when(hasu)
        def _():
            pltpu.async_copy(Ut_hbm.at[:, pl.ds(bb, 128)], slabs[0], sem)

        @pl.when(hasv)
        def _():
            pltpu.async_copy(agt_hbm.at[:, pl.ds(bb, 128)], slabs[1], sem)
            pltpu.async_copy(pet_hbm.at[:, pl.ds(bb, 128)], slabs[2], sem)

    def wait_slabs(b, slabs, sem):
        hasu, hasv = bucket_flags(b)

        @pl.when(hasu)
        def _():
            pltpu.make_async_copy(
                Ut_hbm.at[:, pl.ds(0, 128)], slabs[0], sem).wait()

        @pl.when(hasv)
        def _():
            pltpu.make_async_copy(
                Ut_hbm.at[:, pl.ds(0, 128)], slabs[1], sem).wait()
            pltpu.make_async_copy(
                Ut_hbm.at[:, pl.ds(0, 128)], slabs[2], sem).wait()

    def reinit_fp(fp):
        for j in range(4):
            fp[0, pl.ds(j * L, L)] = B + iota

    def flush_u():
        pltpu.async_copy(fr_u, su_hbm.at[fp_u.at[0]], semF)
        pltpu.make_async_copy(fr_u, su_hbm.at[pl.ds(0, 64)], semF).wait()
        reinit_fp(fp_u)

    def flush_v():
        pltpu.async_copy(fr_a, sa_hbm.at[fp_v.at[0]], semF)
        pltpu.async_copy(fr_p, sp_hbm.at[fp_v.at[0]], semF)
        pltpu.make_async_copy(fr_a, sa_hbm.at[pl.ds(0, 64)], semF).wait()
        pltpu.make_async_copy(fr_p, sp_hbm.at[pl.ds(0, 64)], semF).wait()
        reinit_fp(fp_v)

    def extract(slabs, off, valid_n, ufill, vfill):
        v16 = svals_v[pl.ds(off, L)]
        valid = iota < valid_n
        pos = jnp.bitwise_and(v16, 16383)
        tagv = jnp.bitwise_and(lax.shift_right_logical(v16, 14), 1)
        col = jnp.bitwise_and(lax.shift_right_logical(v16, 15), 127)
        mu = valid & (tagv == 0)
        mv = valid & (tagv == 1)
        ru = plsc.cumsum(jnp.where(mu, 1, 0)) - 1 + ufill
        rv = plsc.cumsum(jnp.where(mv, 1, 0)) - 1 + vfill
        plsc.store_scatter(fp_u, [zeros, ru], pos, mask=mu)
        plsc.store_scatter(fp_v, [zeros, rv], pos, mask=mv)
        for k in range(K):
            kk = jnp.full((L,), k, _i32)
            gu = plsc.load_gather(slabs[0], [kk, col])
            plsc.store_scatter(fr_u, [ru, kk], gu, mask=mu)
            ga = plsc.load_gather(slabs[1], [kk, col])
            plsc.store_scatter(fr_a, [rv, kk], ga, mask=mv)
            gp = plsc.load_gather(slabs[2], [kk, col])
            plsc.store_scatter(fr_p, [rv, kk], gp, mask=mv)
        nu = plsc.all_reduce_population_count(mu)[0]
        nv = plsc.all_reduce_population_count(mv)[0]
        return ufill + nu, vfill + nv

    def process_bucket(b, slabs, fills):
        bounds = starts_v[pl.ds(b - lo, L)]
        s0 = bounds[0]
        nloc = bounds[1] - s0

        def egroup(j, fills):
            ufill, vfill = fills
            ufill, vfill = extract(
                slabs, s0 + j * L, jnp.minimum(nloc - j * L, L), ufill, vfill)
            fu = ufill > 48

            @pl.when(fu)
            def _():
                flush_u()

            ufill = jnp.where(fu, 0, ufill)
            fv = vfill > 48

            @pl.when(fv)
            def _():
                flush_v()

            vfill = jnp.where(fv, 0, vfill)
            return (ufill, vfill)

        negrp = lax.div(nloc + (L - 1), jnp.asarray(L, _i32))
        return lax.fori_loop(0, negrp, egroup, fills)

    slabsA = (sau, saa, sap)
    slabsB = (sbu, sba, sbp)

    reinit_fp(fp_u)
    reinit_fp(fp_v)

    @pl.when(lo < hi)
    def _():
        issue(lo, slabsA, semA)

    @pl.when(lo + 1 < hi)
    def _():
        issue(lo + 1, slabsB, semB)

    def pair_body(i, fills):
        b0 = lo + 2 * i
        b1 = b0 + 1

        wait_slabs(b0, slabsA, semA)
        fills = process_bucket(b0, slabsA, fills)

        @pl.when(b0 + 2 < hi)
        def _():
            issue(b0 + 2, slabsA, semA)

        @pl.when(b1 < hi)
        def _():
            wait_slabs(b1, slabsB, semB)

        fills = process_bucket(b1, slabsB, fills)

        @pl.when(b1 + 2 < hi)
        def _():
            issue(b1 + 2, slabsB, semB)

        return fills

    npairs = lax.div(hi - lo + 1, jnp.asarray(2, _i32))
    z = jnp.asarray(0, _i32)
    ufill, vfill = lax.fori_loop(0, npairs, pair_body, (z, z))

    @pl.when(ufill > 0)
    def _():
        flush_u()

    @pl.when(vfill > 0)
    def _():
        flush_v()


@functools.partial(
    pl.kernel,
    out_type=(
        jax.ShapeDtypeStruct((NW, 4, 128), _f32),
        jax.ShapeDtypeStruct((NW, 4, 128), _f32),
    ),
    mesh=_mesh,
    compiler_params=_params,
    scratch_types=[
        pltpu.VMEM((2, 128, 128), _f32),
        pltpu.VMEM((2, 128, 128), _f32),
        pltpu.VMEM((2, 128, 128), _f32),
        pltpu.VMEM((4, 128), _f32),
        pltpu.VMEM((4, 128), _f32),
        pltpu.SemaphoreType.DMA,
        pltpu.SemaphoreType.DMA,
    ],
)
def _dot_kernel(su_hbm, sa_hbm, sp_hbm, oag_hbm, ope_hbm,
                bu, ba, bp, oag_v, ope_v, sem0, sem1):
    wid = lax.axis_index("s") * NC + lax.axis_index("c")
    base = wid * 512
    iota = lax.iota(_i32, L)
    sems = (sem0, sem1)

    def issue_chunk(c):
        sl = c % 2
        pltpu.async_copy(su_hbm.at[pl.ds(base + c * 128, 128)], bu.at[sl], sems[sl])
        pltpu.async_copy(sa_hbm.at[pl.ds(base + c * 128, 128)], ba.at[sl], sems[sl])
        pltpu.async_copy(sp_hbm.at[pl.ds(base + c * 128, 128)], bp.at[sl], sems[sl])

    def wait_chunk(c):
        sl = c % 2
        pltpu.make_async_copy(su_hbm.at[pl.ds(0, 128)], bu.at[sl], sems[sl]).wait()
        pltpu.make_async_copy(sa_hbm.at[pl.ds(0, 128)], ba.at[sl], sems[sl]).wait()
        pltpu.make_async_copy(sp_hbm.at[pl.ds(0, 128)], bp.at[sl], sems[sl]).wait()

    issue_chunk(0)
    issue_chunk(1)
    for c in range(4):
        sl = c % 2
        wait_chunk(c)

        def grp_body(g, carry, c=c, sl=sl):
            rows = iota + g * L
            acc_a = jnp.zeros((L,), _f32)
            acc_p = jnp.zeros((L,), _f32)
            for k in range(K):
                kk = jnp.full((L,), k, _i32)
                u = plsc.load_gather(bu.at[sl], [rows, kk])
                a = plsc.load_gather(ba.at[sl], [rows, kk])
                p = plsc.load_gather(bp.at[sl], [rows, kk])
                acc_a = acc_a + u * a
                acc_p = acc_p + u * p
            oag_v[c, pl.ds(g * L, L)] = acc_a
            ope_v[c, pl.ds(g * L, L)] = acc_p
            return carry

        lax.fori_loop(0, 8, grp_body, 0)
        if c + 2 < 4:
            issue_chunk(c + 2)

    pltpu.sync_copy(oag_v, oag_hbm.at[wid])
    pltpu.sync_copy(ope_v, ope_hbm.at[wid])


def kernel(user_id, value_id, U, ag_V, pe_V):
    uid2 = user_id.reshape(128, 128)
    vid2 = value_id.reshape(128, 128)
    su, sa, sp = _gather_kernel(uid2, vid2, U.T, ag_V.T, pe_V.T)
    oag, ope = _dot_kernel(su, sa, sp)
    return (oag.reshape(B), ope.reshape(B))
